# baseline (speedup 1.0000x reference)
import jax
import jax.numpy as jnp
from jax.experimental import pallas as pl

_P = 4


def _body(x_ref, wbig_ref, b1big_ref, w2big_ref, b2_ref, o_ref):
    y = jnp.dot(x_ref[...], wbig_ref[...], preferred_element_type=jnp.float32)
    z = jnp.maximum(y + b1big_ref[...], 0.0)
    v = z * w2big_ref[...]
    n_tok = x_ref.shape[0] * _P
    o_ref[...] = jnp.sum(v).reshape(1, 1, 1) + n_tok * b2_ref[...]


def kernel(embs, W1, b1, W2, b2):
    B, L, D = embs.shape
    H = W1.shape[1]
    M = L // _P
    x = embs.reshape(B * M, _P * D)
    wbig = jnp.kron(jnp.eye(_P, dtype=W1.dtype), W1)
    b1big = jnp.tile(b1, _P).reshape(1, _P * H)
    w2big = jnp.tile(W2.reshape(H), _P).reshape(1, _P * H)
    b2r = b2.reshape(1, 1)

    out = pl.pallas_call(
        _body,
        grid=(B,),
        in_specs=[
            pl.BlockSpec((M, _P * D), lambda i: (i, 0)),
            pl.BlockSpec((_P * D, _P * H), lambda i: (0, 0)),
            pl.BlockSpec((1, _P * H), lambda i: (0, 0)),
            pl.BlockSpec((1, _P * H), lambda i: (0, 0)),
            pl.BlockSpec((1, 1), lambda i: (0, 0)),
        ],
        out_specs=pl.BlockSpec((1, 1, 1), lambda i: (i, 0, 0)),
        out_shape=jax.ShapeDtypeStruct((B, 1, 1), jnp.float32),
    )(x, wbig, b1big, w2big, b2r)
    return out.reshape(B)


# speedup vs baseline: 1.6251x; 1.6251x over previous
import jax
import jax.numpy as jnp
from jax.experimental import pallas as pl

_SPB = 4


def _body(x_ref, w1_ref, b1_ref, w2t_ref, b2_ref, o_ref):
    n, d = x_ref.shape
    L = n // _SPB
    h = jnp.dot(x_ref[...], w1_ref[...], preferred_element_type=jnp.float32)
    z = jnp.maximum(h + b1_ref[...], 0.0)
    v = z * w2t_ref[...]
    s = jnp.sum(v.reshape(_SPB, L, v.shape[-1]), axis=(1, 2))
    o_ref[...] = s.reshape(1, 1, _SPB) + L * b2_ref[...]


def kernel(embs, W1, b1, W2, b2):
    B, L, D = embs.shape
    H = W1.shape[1]
    x = embs.reshape(B * L, D)
    b1r = b1.reshape(1, H)
    w2t = W2.reshape(1, H)
    b2r = b2.reshape(1, 1)

    out = pl.pallas_call(
        _body,
        grid=(B // _SPB,),
        in_specs=[
            pl.BlockSpec((_SPB * L, D), lambda i: (i, 0)),
            pl.BlockSpec((D, H), lambda i: (0, 0)),
            pl.BlockSpec((1, H), lambda i: (0, 0)),
            pl.BlockSpec((1, H), lambda i: (0, 0)),
            pl.BlockSpec((1, 1), lambda i: (0, 0)),
        ],
        out_specs=pl.BlockSpec((1, 1, _SPB), lambda i: (i, 0, 0)),
        out_shape=jax.ShapeDtypeStruct((B // _SPB, 1, _SPB), jnp.float32),
    )(x, W1, b1r, w2t, b2r)
    return out.reshape(B)
